# baseline (speedup 1.0000x reference)
import functools

import jax
import jax.numpy as jnp
from jax import lax
from jax.experimental import pallas as pl
from jax.experimental.pallas import tpu as pltpu
from jax.experimental.pallas import tpu_sc as plsc

_CHUNK = 128
_LANES = 16


def _sc_main(v, slen, words, table_pad):
    b_dim, t_dim, d_dim = v.shape
    max_chunks_half = t_dim // _CHUNK // 2
    nd = d_dim // _LANES
    mesh = plsc.VectorSubcoreMesh(core_axis_name="c", subcore_axis_name="s")

    @functools.partial(
        pl.kernel,
        out_type=jax.ShapeDtypeStruct((2, b_dim, 3, d_dim), jnp.float32),
        mesh=mesh,
        scratch_types=[
            pltpu.VMEM((_LANES,), jnp.int32),
            pltpu.VMEM((max_chunks_half, _CHUNK), jnp.int32),
            pltpu.VMEM((max_chunks_half, _CHUNK), jnp.float32),
            pltpu.VMEM((_CHUNK, d_dim), jnp.float32),
            pltpu.VMEM((3, d_dim), jnp.float32),
            pltpu.SemaphoreType.DMA,
        ],
        compiler_params=pltpu.CompilerParams(needs_layout_passes=False),
    )
    def main_kernel(
        v_hbm, slen_hbm, words_hbm, table_hbm, out_hbm,
        len_v, idx_v, w_v, vbuf, pbuf, semw,
    ):
        wid = lax.axis_index("s") * 2 + lax.axis_index("c")
        r = wid // 2
        h = wid % 2

        pltpu.sync_copy(slen_hbm, len_v)
        lvec = len_v[...]
        rows16 = lax.iota(jnp.int32, _LANES)
        lsc = jnp.max(jnp.where(rows16 == r, lvec, 0))
        nc = (lsc + (_CHUNK - 1)) // _CHUNK
        nj = (nc - h + 1) // 2

        for j in range(max_chunks_half):
            @pl.when(j < nj)
            def _():
                c = h + 2 * j
                pltpu.async_copy(
                    words_hbm.at[r, pl.ds(c * _CHUNK, _CHUNK)],
                    idx_v.at[j],
                    semw,
                )
        for j in range(max_chunks_half):
            @pl.when(j < nj)
            def _():
                c = h + 2 * j
                pltpu.make_async_copy(
                    words_hbm.at[r, pl.ds(c * _CHUNK, _CHUNK)],
                    idx_v.at[j],
                    semw,
                ).wait()
        for j in range(max_chunks_half):
            @pl.when(j < nj)
            def _():
                pltpu.async_copy(table_hbm.at[idx_v.at[j]], w_v.at[j], semw)
        for j in range(max_chunks_half):
            @pl.when(j < nj)
            def _():
                pltpu.make_async_copy(
                    table_hbm.at[idx_v.at[j]], w_v.at[j], semw
                ).wait()

        zeros = [jnp.zeros((_LANES,), jnp.float32) for _ in range(3 * nd)]

        def chunk_body(j, acc):
            c = h + 2 * j
            pltpu.sync_copy(v_hbm.at[r, pl.ds(c * _CHUNK, _CHUNK)], vbuf)

            def group_body(g, acc_g):
                t0 = c * _CHUNK + g * _LANES
                wv = w_v[j, pl.ds(g * _LANES, _LANES)]
                lanes = lax.iota(jnp.int32, _LANES)
                accs = list(acc_g)
                for k in range(_LANES):
                    mk = jnp.where(
                        t0 + k < lsc, jnp.float32(1.0), jnp.float32(0.0)
                    )
                    wk = jnp.sum(jnp.where(lanes == k, wv, 0.0)) * mk
                    bm = jnp.broadcast_to(mk, (_LANES,))
                    bw = jnp.broadcast_to(wk, (_LANES,))
                    tok = g * _LANES + k
                    for l in range(nd):
                        vt = vbuf[tok, pl.ds(l * _LANES, _LANES)]
                        vm = vt * bm
                        accs[l] = accs[l] + vm
                        accs[nd + l] = accs[nd + l] + jnp.abs(vm)
                        accs[2 * nd + l] = accs[2 * nd + l] + bw * vt
                return tuple(accs)

            return lax.fori_loop(
                0, _CHUNK // _LANES, group_body, tuple(acc), unroll=False
            )

        acc = lax.fori_loop(0, nj, chunk_body, tuple(zeros), unroll=False)

        for a in range(3):
            for l in range(nd):
                pbuf[a, pl.ds(l * _LANES, _LANES)] = acc[a * nd + l]
        pltpu.sync_copy(pbuf, out_hbm.at[h, r])

    return main_kernel(v, slen, words, table_pad)


def _tc_combine(partials):
    _, b_dim, _, d_dim = partials.shape

    def body(p_ref, y_ref, yh_ref):
        s = p_ref[0] + p_ref[1]
        y_ref[...] = s[:, 0, :] / s[:, 1, :]
        yh_ref[...] = s[:, 2, :]

    return pl.pallas_call(
        body,
        out_shape=[
            jax.ShapeDtypeStruct((b_dim, d_dim), jnp.float32),
            jax.ShapeDtypeStruct((b_dim, d_dim), jnp.float32),
        ],
    )(partials)


def kernel(vector_sequence, sentence_length, word_sequence, W):
    b_dim, t_dim, d_dim = vector_sequence.shape
    vocab = W.shape[0]
    slen = sentence_length.astype(jnp.int32)
    words = word_sequence.astype(jnp.int32)
    vpad = ((vocab + 1023) // 1024) * 1024
    table_pad = jnp.pad(W.astype(jnp.float32), (0, vpad - vocab))
    partials = _sc_main(vector_sequence, slen, words, table_pad)
    y, y_hat = _tc_combine(partials)
    return (y, y_hat)
